# baseline (speedup 1.0000x reference)
import jax
import jax.numpy as jnp
from jax import lax
from jax.experimental import pallas as pl
from jax.experimental.pallas import tpu as pltpu

_ROWS = 32
_COLS = 1_000_000
_SUB = 8
_LANES = _COLS // _SUB

_ROT_A = (13, 15, 26, 6)
_ROT_B = (17, 29, 16, 24)


def _threefry_bits(x0, x1):
    ks = (jnp.uint32(0), jnp.uint32(1), jnp.uint32(0x1BD11BDA ^ 0 ^ 1))
    x0 = x0 + ks[0]
    x1 = x1 + ks[1]
    for i in range(5):
        for r in (_ROT_A if i % 2 == 0 else _ROT_B):
            x0 = x0 + x1
            x1 = (x1 << r) | (x1 >> (32 - r))
            x1 = x1 ^ x0
        x0 = x0 + ks[(i + 1) % 3]
        x1 = x1 + ks[(i + 2) % 3] + jnp.uint32(i + 1)
    return x0 ^ x1


def _gumbel_softmax_kernel(inv_t_ref, x_ref, o_ref):
    row = pl.program_id(0)
    sub = lax.broadcasted_iota(jnp.uint32, (_SUB, _LANES), 0)
    lane = lax.broadcasted_iota(jnp.uint32, (_SUB, _LANES), 1)
    idx = jnp.uint32(row * _COLS) + sub * jnp.uint32(_LANES) + lane
    bits = _threefry_bits(jnp.zeros((_SUB, _LANES), jnp.uint32), idx)
    fbits = (bits >> 9) | jnp.uint32(0x3F800000)
    u = lax.bitcast_convert_type(fbits, jnp.float32) - jnp.float32(1.0)
    eps = jnp.float32(1e-20)
    g = -jnp.log(-jnp.log(u + eps) + eps)
    z = (x_ref[0] + g) * inv_t_ref[0, 0]
    m = jnp.max(z)
    e = jnp.exp(z - m)
    s = jnp.sum(e)
    o_ref[0] = e * (jnp.float32(1.0) / s)


def kernel(logits, temperature, use_gpu):
    del use_gpu
    inv_t = (jnp.float32(1.0)
             / jnp.asarray(temperature, jnp.float32)).reshape(1, 1)
    out = pl.pallas_call(
        _gumbel_softmax_kernel,
        grid=(_ROWS,),
        in_specs=[
            pl.BlockSpec(memory_space=pltpu.SMEM),
            pl.BlockSpec((1, _SUB, _LANES), lambda i: (i, 0, 0)),
        ],
        out_specs=pl.BlockSpec((1, _SUB, _LANES), lambda i: (i, 0, 0)),
        out_shape=jax.ShapeDtypeStruct((_ROWS, _SUB, _LANES), jnp.float32),
        compiler_params=pltpu.CompilerParams(
            dimension_semantics=("parallel",),
        ),
    )(inv_t, logits.reshape(_ROWS, _SUB, _LANES))
    return out.reshape(_ROWS, _COLS)


# speedup vs baseline: 1.0981x; 1.0981x over previous
import jax
import jax.numpy as jnp
from jax import lax
from jax.experimental import pallas as pl
from jax.experimental.pallas import tpu as pltpu

_ROWS = 32
_COLS = 1_000_000
_S = 1000
_L = 1000
_CH = 8
_NCH = _S // _CH

_ROT_A = (13, 15, 26, 6)
_ROT_B = (17, 29, 16, 24)
_KS = (0, 1, 0x1BD11BDA ^ 0 ^ 1)


def _threefry_bits(x1):
    x0 = x1
    x1 = ((x1 << 13) | (x1 >> 19)) ^ x0
    first = True
    for i in range(5):
        rots = _ROT_A if i % 2 == 0 else _ROT_B
        for r in (rots[1:] if first else rots):
            x0 = x0 + x1
            x1 = (x1 << r) | (x1 >> (32 - r))
            x1 = x1 ^ x0
        first = False
        x0 = x0 + jnp.uint32(_KS[(i + 1) % 3])
        x1 = x1 + jnp.uint32(_KS[(i + 2) % 3] + i + 1)
    return x0 ^ x1


def _gumbel_softmax_kernel(inv_t_ref, x_ref, o_ref):
    row = pl.program_id(0)
    inv_t = inv_t_ref[0, 0]
    eps = jnp.float32(1e-20)
    sub = lax.broadcasted_iota(jnp.uint32, (_CH, _L), 0)
    lane = lax.broadcasted_iota(jnp.uint32, (_CH, _L), 1)
    cvec = sub * jnp.uint32(_L) + lane
    base = jnp.uint32(row * _COLS + 1)

    def z_body(k, m_vec):
        off = (k * (_CH * _L)).astype(jnp.uint32) + base
        bits = _threefry_bits(cvec + off)
        fbits = (bits >> 9) | jnp.uint32(0x3F800000)
        u = lax.bitcast_convert_type(fbits, jnp.float32) - jnp.float32(1.0)
        g = -jnp.log(-jnp.log(u + eps) + eps)
        z = (x_ref[0, pl.ds(k * _CH, _CH), :] + g) * inv_t
        o_ref[0, pl.ds(k * _CH, _CH), :] = z
        return jnp.maximum(m_vec, z)

    m_vec = lax.fori_loop(
        0, _NCH, z_body, jnp.full((_CH, _L), -jnp.inf, jnp.float32))
    m = jnp.max(m_vec)

    def e_body(k, s_vec):
        e = jnp.exp(o_ref[0, pl.ds(k * _CH, _CH), :] - m)
        o_ref[0, pl.ds(k * _CH, _CH), :] = e
        return s_vec + e

    s_vec = lax.fori_loop(
        0, _NCH, e_body, jnp.zeros((_CH, _L), jnp.float32))
    inv_s = jnp.float32(1.0) / jnp.sum(s_vec)

    def scale_body(k, carry):
        o_ref[0, pl.ds(k * _CH, _CH), :] *= inv_s
        return carry

    lax.fori_loop(0, _NCH, scale_body, jnp.float32(0.0))


def kernel(logits, temperature, use_gpu):
    del use_gpu
    inv_t = (jnp.float32(1.0)
             / jnp.asarray(temperature, jnp.float32)).reshape(1, 1)
    out = pl.pallas_call(
        _gumbel_softmax_kernel,
        grid=(_ROWS,),
        in_specs=[
            pl.BlockSpec(memory_space=pltpu.SMEM),
            pl.BlockSpec((1, _S, _L), lambda i: (i, 0, 0)),
        ],
        out_specs=pl.BlockSpec((1, _S, _L), lambda i: (i, 0, 0)),
        out_shape=jax.ShapeDtypeStruct((_ROWS, _S, _L), jnp.float32),
        compiler_params=pltpu.CompilerParams(
            dimension_semantics=("parallel",),
        ),
    )(inv_t, logits.reshape(_ROWS, _S, _L))
    return out.reshape(_ROWS, _COLS)


# speedup vs baseline: 1.2534x; 1.1414x over previous
import jax
import jax.numpy as jnp
from jax import lax
from jax.experimental import pallas as pl
from jax.experimental.pallas import tpu as pltpu

_ROWS = 32
_COLS = 1_000_000
_S = 1000
_L = 1000
_CH = 40
_NCH = _S // _CH

_ROT_A = (13, 15, 26, 6)
_ROT_B = (17, 29, 16, 24)
_KS = (0, 1, 0x1BD11BDA ^ 0 ^ 1)


def _threefry_bits(x1):
    x0 = x1
    x1 = ((x1 << 13) | (x1 >> 19)) ^ x0
    first = True
    for i in range(5):
        rots = _ROT_A if i % 2 == 0 else _ROT_B
        for r in (rots[1:] if first else rots):
            x0 = x0 + x1
            x1 = (x1 << r) | (x1 >> (32 - r))
            x1 = x1 ^ x0
        first = False
        x0 = x0 + jnp.uint32(_KS[(i + 1) % 3])
        x1 = x1 + jnp.uint32(_KS[(i + 2) % 3] + i + 1)
    return x0 ^ x1


def _gumbel_softmax_kernel(inv_t_ref, x_ref, o_ref):
    row = pl.program_id(0)
    inv_t = inv_t_ref[0, 0]
    eps = jnp.float32(1e-20)
    sub = lax.broadcasted_iota(jnp.uint32, (_CH, _L), 0)
    lane = lax.broadcasted_iota(jnp.uint32, (_CH, _L), 1)
    cvec = sub * jnp.uint32(_L) + lane
    base = jnp.uint32(row * _COLS + 1)

    def z_body(k, m_vec):
        off = (k * (_CH * _L)).astype(jnp.uint32) + base
        bits = _threefry_bits(cvec + off)
        fbits = (bits >> 9) | jnp.uint32(0x3F800000)
        u = lax.bitcast_convert_type(fbits, jnp.float32) - jnp.float32(1.0)
        g = -jnp.log(-jnp.log(u + eps) + eps)
        z = (x_ref[0, pl.ds(k * _CH, _CH), :] + g) * inv_t
        o_ref[0, pl.ds(k * _CH, _CH), :] = z
        return jnp.maximum(m_vec, z)

    m_vec = lax.fori_loop(
        0, _NCH, z_body, jnp.full((_CH, _L), -jnp.inf, jnp.float32))
    m = jnp.max(m_vec)

    def e_body(k, s_vec):
        e = jnp.exp(o_ref[0, pl.ds(k * _CH, _CH), :] - m)
        o_ref[0, pl.ds(k * _CH, _CH), :] = e
        return s_vec + e

    s_vec = lax.fori_loop(
        0, _NCH, e_body, jnp.zeros((_CH, _L), jnp.float32))
    inv_s = jnp.float32(1.0) / jnp.sum(s_vec)

    def scale_body(k, carry):
        o_ref[0, pl.ds(k * _CH, _CH), :] *= inv_s
        return carry

    lax.fori_loop(0, _NCH, scale_body, jnp.float32(0.0))


def kernel(logits, temperature, use_gpu):
    del use_gpu
    inv_t = (jnp.float32(1.0)
             / jnp.asarray(temperature, jnp.float32)).reshape(1, 1)
    out = pl.pallas_call(
        _gumbel_softmax_kernel,
        grid=(_ROWS,),
        in_specs=[
            pl.BlockSpec(memory_space=pltpu.SMEM),
            pl.BlockSpec((1, _S, _L), lambda i: (i, 0, 0)),
        ],
        out_specs=pl.BlockSpec((1, _S, _L), lambda i: (i, 0, 0)),
        out_shape=jax.ShapeDtypeStruct((_ROWS, _S, _L), jnp.float32),
        compiler_params=pltpu.CompilerParams(
            dimension_semantics=("parallel",),
        ),
    )(inv_t, logits.reshape(_ROWS, _S, _L))
    return out.reshape(_ROWS, _COLS)
